# baseline (speedup 1.0000x reference)
import functools

import jax
import jax.numpy as jnp
from jax import lax
from jax.experimental import pallas as pl
from jax.experimental.pallas import tpu as pltpu
from jax.experimental.pallas import tpu_sc as plsc

N = 10000
E = 320000
D = 128
NC = 2
NS = 16
NW = NC * NS
WIN = 128
WT = E // WIN
WBASE = 78
WMAIN = 76
ROWS_PER_TILE = 1000

_MESH = plsc.VectorSubcoreMesh(core_axis_name="c", subcore_axis_name="s")


def _deg_body(col_hbm, zeros_hbm, cnt_hbm, colv, ones_v, deg_sh,
              i0, i1, i2, i3):
    cid = lax.axis_index("c")
    sid = lax.axis_index("s")
    wid = sid * NC + cid
    isems = (i0, i1, i2, i3)
    base = WBASE * wid + jnp.minimum(wid, 4)
    nw = jnp.where(wid < 4, WBASE + 1, WBASE)

    @pl.when(sid == 0)
    def _():
        pltpu.sync_copy(zeros_hbm, deg_sh)

    for i in range(8):
        ones_v[pl.ds(i * 16, 16)] = jnp.ones((16,), jnp.float32)
    plsc.subcore_barrier()

    for s in range(4):
        pltpu.async_copy(col_hbm.at[base + s], colv.at[s], isems[s])

    def body(wg, carry):
        for j in range(4):
            w = wg * 4 + j
            pltpu.make_async_copy(col_hbm.at[base + w], colv.at[j],
                                  isems[j]).wait()
            pltpu.sync_copy(ones_v, deg_sh.at[colv.at[j]], add=True)

            @pl.when(w + 4 < nw)
            def _():
                pltpu.async_copy(col_hbm.at[base + w + 4], colv.at[j],
                                 isems[j])

        return carry

    lax.fori_loop(0, WMAIN // 4, body, 0)
    for j, w in ((0, WMAIN), (1, WMAIN + 1)):
        pltpu.make_async_copy(col_hbm.at[base + w], colv.at[j],
                              isems[j]).wait()
        pltpu.sync_copy(ones_v, deg_sh.at[colv.at[j]], add=True)

    @pl.when(nw > WBASE)
    def _():
        pltpu.make_async_copy(col_hbm.at[base + WBASE], colv.at[2],
                              isems[2]).wait()
        pltpu.sync_copy(ones_v, deg_sh.at[colv.at[2]], add=True)

    plsc.subcore_barrier()

    @pl.when(sid == 0)
    def _():
        pltpu.sync_copy(deg_sh, cnt_hbm.at[cid])


_deg = pl.kernel(
    _deg_body,
    out_type=jax.ShapeDtypeStruct((NC, N), jnp.float32),
    mesh=_MESH,
    scratch_types=[
        pltpu.VMEM((4, WIN), jnp.int32),
        pltpu.VMEM((128,), jnp.float32),
        pltpu.VMEM_SHARED((N,), jnp.float32),
        pltpu.SemaphoreType.DMA,
        pltpu.SemaphoreType.DMA,
        pltpu.SemaphoreType.DMA,
        pltpu.SemaphoreType.DMA,
    ],
)


def _scat_body(g_hbm, row_hbm, col_hbm, zeros_hbm, out_hbm,
               idxr, b0, b1, acc_sh, i0, i1, i2, i3, g0, g1):
    cid = lax.axis_index("c")
    sid = lax.axis_index("s")
    wid = sid * NC + cid
    bufs = (b0, b1)
    gsems = (g0, g1)
    isems = (i0, i1, i2, i3)

    base = WBASE * wid + jnp.minimum(wid, 4)
    nw = jnp.where(wid < 4, WBASE + 1, WBASE)

    r0 = sid * ROWS_PER_TILE

    @pl.when(sid < N // ROWS_PER_TILE)
    def _():
        @pl.when(cid == 0)
        def _():
            pltpu.sync_copy(g_hbm.at[pl.ds(r0, ROWS_PER_TILE)],
                            acc_sh.at[pl.ds(r0, ROWS_PER_TILE)])

        @pl.when(cid == 1)
        def _():
            pltpu.sync_copy(zeros_hbm.at[pl.ds(r0, ROWS_PER_TILE)],
                            acc_sh.at[pl.ds(r0, ROWS_PER_TILE)])

    plsc.subcore_barrier()

    def rows(s):
        return idxr.at[2 * s]

    def cols(s):
        return idxr.at[2 * s + 1]

    def idx_fetch(w, s, sem):
        pltpu.async_copy(row_hbm.at[base + w], idxr.at[2 * s], sem)
        pltpu.async_copy(col_hbm.at[base + w], idxr.at[2 * s + 1], sem)

    def idx_wait(w, s, sem):
        pltpu.make_async_copy(row_hbm.at[base + w], idxr.at[2 * s],
                              sem).wait()
        pltpu.make_async_copy(col_hbm.at[base + w], idxr.at[2 * s + 1],
                              sem).wait()

    for s in range(4):
        idx_fetch(s, s, isems[s])
    for k in range(2):
        idx_wait(k, k, isems[k])
        pltpu.async_copy(g_hbm.at[cols(k)], bufs[k], gsems[k])

    def body(wg, carry):
        for j in range(4):
            w = wg * 4 + j
            b = j % 2
            sj = (j + 2) % 4
            pltpu.make_async_copy(g_hbm.at[cols(j)], bufs[b],
                                  gsems[b]).wait()
            pltpu.sync_copy(bufs[b], acc_sh.at[rows(j)], add=True)
            idx_wait(w + 2, sj, isems[sj])
            pltpu.async_copy(g_hbm.at[cols(sj)], bufs[b], gsems[b])

            @pl.when(w + 4 < nw)
            def _():
                idx_fetch(w + 4, j, isems[j])

        return carry

    lax.fori_loop(0, WMAIN // 4, body, 0)
    pltpu.make_async_copy(g_hbm.at[cols(0)], bufs[0], gsems[0]).wait()
    pltpu.sync_copy(bufs[0], acc_sh.at[rows(0)], add=True)

    @pl.when(nw > WBASE)
    def _():
        idx_wait(WBASE, 2, isems[2])
        pltpu.async_copy(g_hbm.at[cols(2)], bufs[0], gsems[0])

    pltpu.make_async_copy(g_hbm.at[cols(1)], bufs[1], gsems[1]).wait()
    pltpu.sync_copy(bufs[1], acc_sh.at[rows(1)], add=True)

    @pl.when(nw > WBASE)
    def _():
        pltpu.make_async_copy(g_hbm.at[cols(2)], bufs[0], gsems[0]).wait()
        pltpu.sync_copy(bufs[0], acc_sh.at[rows(2)], add=True)

    plsc.subcore_barrier()

    @pl.when(sid < N // ROWS_PER_TILE)
    def _():
        pltpu.sync_copy(acc_sh.at[pl.ds(r0, ROWS_PER_TILE)],
                        out_hbm.at[cid, pl.ds(r0, ROWS_PER_TILE)])


_scat = pl.kernel(
    _scat_body,
    out_type=jax.ShapeDtypeStruct((NC, N, D), jnp.float32),
    mesh=_MESH,
    scratch_types=[
        pltpu.VMEM((8, 128), jnp.int32),
        pltpu.VMEM((WIN, D), jnp.float32),
        pltpu.VMEM((WIN, D), jnp.float32),
        pltpu.VMEM_SHARED((N, D), jnp.float32),
        pltpu.SemaphoreType.DMA,
        pltpu.SemaphoreType.DMA,
        pltpu.SemaphoreType.DMA,
        pltpu.SemaphoreType.DMA,
        pltpu.SemaphoreType.DMA,
        pltpu.SemaphoreType.DMA,
    ],
)


BR = 2000


def _lin_body(x_ref, wt_ref, b_ref, h_ref):
    h_ref[...] = jnp.dot(x_ref[...], wt_ref[...],
                         preferred_element_type=jnp.float32) + b_ref[...]


_lin = pl.pallas_call(
    _lin_body,
    grid=(N // BR,),
    in_specs=[
        pl.BlockSpec((BR, D), lambda i: (i, 0)),
        pl.BlockSpec((D, D), lambda i: (0, 0)),
        pl.BlockSpec((1, D), lambda i: (0, 0)),
    ],
    out_specs=pl.BlockSpec((BR, D), lambda i: (i, 0)),
    out_shape=jax.ShapeDtypeStruct((N, D), jnp.float32),
)


def _scale_body(cnt_ref, h_ref, g_ref):
    deg = cnt_ref[:, 0:1] + cnt_ref[:, 1:2] + 1.0
    g_ref[...] = h_ref[...] * lax.rsqrt(deg)


_scale = pl.pallas_call(
    _scale_body,
    grid=(N // BR,),
    in_specs=[
        pl.BlockSpec((BR, 2), lambda i: (i, 0)),
        pl.BlockSpec((BR, D), lambda i: (i, 0)),
    ],
    out_specs=pl.BlockSpec((BR, D), lambda i: (i, 0)),
    out_shape=jax.ShapeDtypeStruct((N, D), jnp.float32),
)


def _final_body(cnt_ref, p_ref, o_ref):
    deg = cnt_ref[:, 0:1] + cnt_ref[:, 1:2] + 1.0
    dis = lax.rsqrt(deg)
    p = p_ref[...]
    o_ref[...] = dis * (p[0] + p[1])


_final = pl.pallas_call(
    _final_body,
    grid=(N // BR,),
    in_specs=[
        pl.BlockSpec((BR, 2), lambda i: (i, 0)),
        pl.BlockSpec((NC, BR, D), lambda i: (0, i, 0)),
    ],
    out_specs=pl.BlockSpec((BR, D), lambda i: (i, 0)),
    out_shape=jax.ShapeDtypeStruct((N, D), jnp.float32),
)


def kernel(x, edge_index, W, b):
    ei32 = edge_index.astype(jnp.int32)
    row2 = ei32[0].reshape(WT, WIN)
    col2 = ei32[1].reshape(WT, WIN)
    zeros1 = jnp.zeros((N,), jnp.float32)
    zeros2 = jnp.zeros((N, D), jnp.float32)

    counts = _deg(col2, zeros1)
    cnt_t = counts.T
    h = _lin(x, W.T, b.reshape(1, D))
    g = _scale(cnt_t, h)
    P = _scat(g, row2, col2, zeros2)
    return _final(cnt_t, P)


# speedup vs baseline: 1.0167x; 1.0167x over previous
import functools

import jax
import jax.numpy as jnp
from jax import lax
from jax.experimental import pallas as pl
from jax.experimental.pallas import tpu as pltpu
from jax.experimental.pallas import tpu_sc as plsc

N = 10000
E = 320000
D = 128
NC = 2
NS = 16
NW = NC * NS
WIN = 128
WT = E // WIN
WBASE = 78
WMAIN = 76
EPW = E // NW
WIN_D = 125
NWIN_D = EPW // WIN_D
ROWS_PER_TILE = 1000

_MESH = plsc.VectorSubcoreMesh(core_axis_name="c", subcore_axis_name="s")


def _deg_body(col_hbm, zeros_hbm, cnt_hbm, colv, ones_v, deg_sh, sem):
    cid = lax.axis_index("c")
    sid = lax.axis_index("s")
    wid = sid * NC + cid

    @pl.when(sid == 0)
    def _():
        pltpu.sync_copy(zeros_hbm, deg_sh)

    for i in range(8):
        ones_v[pl.ds(i * 16, 16)] = jnp.ones((16,), jnp.float32)
    pltpu.sync_copy(col_hbm.at[wid], colv)
    plsc.subcore_barrier()

    def body(w, carry):
        pltpu.sync_copy(ones_v.at[pl.ds(0, WIN_D)], deg_sh.at[colv.at[w]],
                        add=True)
        return carry

    lax.fori_loop(0, NWIN_D, body, 0)
    plsc.subcore_barrier()

    @pl.when(sid == 0)
    def _():
        pltpu.sync_copy(deg_sh, cnt_hbm.at[cid])


_deg = pl.kernel(
    _deg_body,
    out_type=jax.ShapeDtypeStruct((NC, N), jnp.float32),
    mesh=_MESH,
    scratch_types=[
        pltpu.VMEM((NWIN_D, WIN_D), jnp.int32),
        pltpu.VMEM((128,), jnp.float32),
        pltpu.VMEM_SHARED((N,), jnp.float32),
        pltpu.SemaphoreType.DMA,
    ],
)


def _scat_body(g_hbm, row_hbm, col_hbm, zeros_hbm, out_hbm,
               idxr, b0, b1, acc_sh, i0, i1, i2, i3, g0, g1):
    cid = lax.axis_index("c")
    sid = lax.axis_index("s")
    wid = sid * NC + cid
    bufs = (b0, b1)
    gsems = (g0, g1)
    isems = (i0, i1, i2, i3)

    base = WBASE * wid + jnp.minimum(wid, 4)
    nw = jnp.where(wid < 4, WBASE + 1, WBASE)

    r0 = sid * ROWS_PER_TILE

    @pl.when(sid < N // ROWS_PER_TILE)
    def _():
        @pl.when(cid == 0)
        def _():
            pltpu.sync_copy(g_hbm.at[pl.ds(r0, ROWS_PER_TILE)],
                            acc_sh.at[pl.ds(r0, ROWS_PER_TILE)])

        @pl.when(cid == 1)
        def _():
            pltpu.sync_copy(zeros_hbm.at[pl.ds(r0, ROWS_PER_TILE)],
                            acc_sh.at[pl.ds(r0, ROWS_PER_TILE)])

    plsc.subcore_barrier()

    def rows(s):
        return idxr.at[2 * s]

    def cols(s):
        return idxr.at[2 * s + 1]

    def idx_fetch(w, s, sem):
        pltpu.async_copy(row_hbm.at[base + w], idxr.at[2 * s], sem)
        pltpu.async_copy(col_hbm.at[base + w], idxr.at[2 * s + 1], sem)

    def idx_wait(w, s, sem):
        pltpu.make_async_copy(row_hbm.at[base + w], idxr.at[2 * s],
                              sem).wait()
        pltpu.make_async_copy(col_hbm.at[base + w], idxr.at[2 * s + 1],
                              sem).wait()

    for s in range(4):
        idx_fetch(s, s, isems[s])
    for k in range(2):
        idx_wait(k, k, isems[k])
        pltpu.async_copy(g_hbm.at[cols(k)], bufs[k], gsems[k])

    def body(wg, carry):
        for j in range(4):
            w = wg * 4 + j
            b = j % 2
            sj = (j + 2) % 4
            pltpu.make_async_copy(g_hbm.at[cols(j)], bufs[b],
                                  gsems[b]).wait()
            pltpu.sync_copy(bufs[b], acc_sh.at[rows(j)], add=True)
            idx_wait(w + 2, sj, isems[sj])
            pltpu.async_copy(g_hbm.at[cols(sj)], bufs[b], gsems[b])

            @pl.when(w + 4 < nw)
            def _():
                idx_fetch(w + 4, j, isems[j])

        return carry

    lax.fori_loop(0, WMAIN // 4, body, 0)
    pltpu.make_async_copy(g_hbm.at[cols(0)], bufs[0], gsems[0]).wait()
    pltpu.sync_copy(bufs[0], acc_sh.at[rows(0)], add=True)

    @pl.when(nw > WBASE)
    def _():
        idx_wait(WBASE, 2, isems[2])
        pltpu.async_copy(g_hbm.at[cols(2)], bufs[0], gsems[0])

    pltpu.make_async_copy(g_hbm.at[cols(1)], bufs[1], gsems[1]).wait()
    pltpu.sync_copy(bufs[1], acc_sh.at[rows(1)], add=True)

    @pl.when(nw > WBASE)
    def _():
        pltpu.make_async_copy(g_hbm.at[cols(2)], bufs[0], gsems[0]).wait()
        pltpu.sync_copy(bufs[0], acc_sh.at[rows(2)], add=True)

    plsc.subcore_barrier()

    @pl.when(sid < N // ROWS_PER_TILE)
    def _():
        pltpu.sync_copy(acc_sh.at[pl.ds(r0, ROWS_PER_TILE)],
                        out_hbm.at[cid, pl.ds(r0, ROWS_PER_TILE)])


_scat = pl.kernel(
    _scat_body,
    out_type=jax.ShapeDtypeStruct((NC, N, D), jnp.float32),
    mesh=_MESH,
    scratch_types=[
        pltpu.VMEM((8, 128), jnp.int32),
        pltpu.VMEM((WIN, D), jnp.float32),
        pltpu.VMEM((WIN, D), jnp.float32),
        pltpu.VMEM_SHARED((N, D), jnp.float32),
        pltpu.SemaphoreType.DMA,
        pltpu.SemaphoreType.DMA,
        pltpu.SemaphoreType.DMA,
        pltpu.SemaphoreType.DMA,
        pltpu.SemaphoreType.DMA,
        pltpu.SemaphoreType.DMA,
    ],
)


BR = 2000


def _lin_body(x_ref, wt_ref, b_ref, h_ref):
    h_ref[...] = jnp.dot(x_ref[...], wt_ref[...],
                         preferred_element_type=jnp.float32) + b_ref[...]


_lin = pl.pallas_call(
    _lin_body,
    grid=(N // BR,),
    in_specs=[
        pl.BlockSpec((BR, D), lambda i: (i, 0)),
        pl.BlockSpec((D, D), lambda i: (0, 0)),
        pl.BlockSpec((1, D), lambda i: (0, 0)),
    ],
    out_specs=pl.BlockSpec((BR, D), lambda i: (i, 0)),
    out_shape=jax.ShapeDtypeStruct((N, D), jnp.float32),
)


def _scale_body(cnt_ref, h_ref, g_ref):
    deg = cnt_ref[:, 0:1] + cnt_ref[:, 1:2] + 1.0
    g_ref[...] = h_ref[...] * lax.rsqrt(deg)


_scale = pl.pallas_call(
    _scale_body,
    grid=(N // BR,),
    in_specs=[
        pl.BlockSpec((BR, 2), lambda i: (i, 0)),
        pl.BlockSpec((BR, D), lambda i: (i, 0)),
    ],
    out_specs=pl.BlockSpec((BR, D), lambda i: (i, 0)),
    out_shape=jax.ShapeDtypeStruct((N, D), jnp.float32),
)


def _final_body(cnt_ref, p_ref, o_ref):
    deg = cnt_ref[:, 0:1] + cnt_ref[:, 1:2] + 1.0
    dis = lax.rsqrt(deg)
    p = p_ref[...]
    o_ref[...] = dis * (p[0] + p[1])


_final = pl.pallas_call(
    _final_body,
    grid=(N // BR,),
    in_specs=[
        pl.BlockSpec((BR, 2), lambda i: (i, 0)),
        pl.BlockSpec((NC, BR, D), lambda i: (0, i, 0)),
    ],
    out_specs=pl.BlockSpec((BR, D), lambda i: (i, 0)),
    out_shape=jax.ShapeDtypeStruct((N, D), jnp.float32),
)


def kernel(x, edge_index, W, b):
    ei32 = edge_index.astype(jnp.int32)
    row2 = ei32[0].reshape(WT, WIN)
    col2 = ei32[1].reshape(WT, WIN)
    col_d = ei32[1].reshape(NW, NWIN_D, WIN_D)
    zeros1 = jnp.zeros((N,), jnp.float32)
    zeros2 = jnp.zeros((N, D), jnp.float32)

    counts = _deg(col_d, zeros1)
    cnt_t = counts.T
    h = _lin(x, W.T, b.reshape(1, D))
    g = _scale(cnt_t, h)
    P = _scat(g, row2, col2, zeros2)
    return _final(cnt_t, P)
